# baseline (speedup 1.0000x reference)
import jax
import jax.numpy as jnp
from jax.experimental import pallas as pl
from jax.experimental.pallas import tpu as pltpu

_D_MODEL = 4096
_VOCAB = 100000
_BATCH = 128
_S = 2
_BN = 512
_W_STEP = _S * _BN


def _sample_kernel(*refs):
    obs_ref = refs[0]
    w_refs = refs[1:1 + _S]
    b_ref = refs[1 + _S]
    g_ref = refs[2 + _S]
    idx_out_ref = refs[3 + _S]
    bestv_ref, besti_ref = refs[4 + _S], refs[5 + _S]

    j = pl.program_id(0)
    nblk = pl.num_programs(0)

    obs = obs_ref[:]
    logits = jnp.concatenate(
        [jnp.dot(obs, w_refs[s][:], preferred_element_type=jnp.float32)
         for s in range(_S)], axis=1) + b_ref[:]
    score = g_ref[:] + logits

    col = jax.lax.broadcasted_iota(jnp.int32, (_BATCH, _W_STEP), 1) + j * _W_STEP
    score = jnp.where(col < _VOCAB, score, -jnp.inf)

    local_max = jnp.max(score, axis=1, keepdims=True)
    local_arg = jnp.min(jnp.where(score == local_max, col, _VOCAB),
                        axis=1, keepdims=True).astype(jnp.int32)

    @pl.when(j == 0)
    def _():
        bestv_ref[:] = local_max
        besti_ref[:] = local_arg

    @pl.when(j > 0)
    def _():
        better = local_max > bestv_ref[:]
        bestv_ref[:] = jnp.where(better, local_max, bestv_ref[:])
        besti_ref[:] = jnp.where(better, local_arg, besti_ref[:])

    @pl.when(j == nblk - 1)
    def _():
        idx_out_ref[:] = besti_ref[:]


def _w_spec(s):
    return pl.BlockSpec((_D_MODEL, _BN), lambda j, s=s: (0, _S * j + s))


def kernel(obs, W, b):
    g = jax.random.gumbel(jax.random.key(42), (_BATCH, _VOCAB), jnp.float32)
    grid = pl.cdiv(_VOCAB, _W_STEP)
    idx = pl.pallas_call(
        _sample_kernel,
        grid=(grid,),
        in_specs=[
            pl.BlockSpec((_BATCH, _D_MODEL), lambda j: (0, 0)),
            *[_w_spec(s) for s in range(_S)],
            pl.BlockSpec((1, _W_STEP), lambda j: (0, j)),
            pl.BlockSpec((_BATCH, _W_STEP), lambda j: (0, j)),
        ],
        out_specs=pl.BlockSpec((_BATCH, 1), lambda j: (0, 0)),
        out_shape=jax.ShapeDtypeStruct((_BATCH, 1), jnp.int32),
        scratch_shapes=[
            pltpu.VMEM((_BATCH, 1), jnp.float32),
            pltpu.VMEM((_BATCH, 1), jnp.int32),
        ],
    )(obs, *([W] * _S), b.reshape(1, _VOCAB), g)
    return idx.reshape(_BATCH)
